# Initial kernel scaffold; baseline (speedup 1.0000x reference)
import functools

import jax
import jax.numpy as jnp
from jax import lax
from jax.experimental import pallas as pl
from jax.experimental.pallas import tpu as pltpu
from jax.experimental.pallas import tpu_sc as plsc

L = 16
NC = 2
NS = 16
NW = NC * NS

P = 262144
DIM = 64
ORDER = 4
NSEG = 1024
ROW = ORDER * DIM

PW = P // NW
C = 128
NCHUNK = PW // C


def _sc_body(table_hbm, xq_hbm, out_hbm, xq_v, idx_v, t_v, rows_v, out_v, sem):
    wid = lax.axis_index("s") * NC + lax.axis_index("c")
    base = wid * PW

    def chunk_body(k, _):
        off = base + k * C
        pltpu.sync_copy(xq_hbm.at[pl.ds(off, C)], xq_v)
        for v in range(C // L):
            x = xq_v[pl.ds(v * L, L)]
            ix = jnp.clip(x.astype(jnp.int32), 0, NSEG - 1)
            idx_v[pl.ds(v * L, L)] = ix
            t_v[pl.ds(v * L, L)] = x - ix.astype(jnp.float32)
        pltpu.async_copy(table_hbm.at[idx_v], rows_v, sem).wait()

        def pt_body(p, _):
            t = t_v[p]
            for q in range(DIM // L):
                c0 = rows_v[p, pl.ds(0 * DIM + q * L, L)]
                c1 = rows_v[p, pl.ds(1 * DIM + q * L, L)]
                c2 = rows_v[p, pl.ds(2 * DIM + q * L, L)]
                c3 = rows_v[p, pl.ds(3 * DIM + q * L, L)]
                out_v[p, pl.ds(q * L, L)] = ((c0 * t + c1) * t + c2) * t + c3
            return 0

        lax.fori_loop(0, C, pt_body, 0)
        pltpu.sync_copy(out_v, out_hbm.at[pl.ds(off, C)])
        return 0

    lax.fori_loop(0, NCHUNK, chunk_body, 0)


@functools.partial(
    pl.kernel,
    mesh=plsc.VectorSubcoreMesh(core_axis_name="c", subcore_axis_name="s"),
    out_type=jax.ShapeDtypeStruct((P, DIM), jnp.float32),
    scratch_types=[
        pltpu.VMEM((C,), jnp.float32),
        pltpu.VMEM((C,), jnp.int32),
        pltpu.VMEM((C,), jnp.float32),
        pltpu.VMEM((C, ROW), jnp.float32),
        pltpu.VMEM((C, DIM), jnp.float32),
        pltpu.SemaphoreType.DMA,
    ],
)
def _sc_ppoly(table_hbm, xq_hbm, out_hbm, xq_v, idx_v, t_v, rows_v, out_v, sem):
    _sc_body(table_hbm, xq_hbm, out_hbm, xq_v, idx_v, t_v, rows_v, out_v, sem)


def kernel(c, x_breaks, xq, i, j):
    del x_breaks
    table = jnp.transpose(c[:, :, i, j, :], (1, 0, 2)).reshape(NSEG, ROW)
    return _sc_ppoly(table, xq)


# speedup vs baseline: 38.6168x; 38.6168x over previous
import functools

import jax
import jax.numpy as jnp
from jax import lax
from jax.experimental import pallas as pl
from jax.experimental.pallas import tpu as pltpu
from jax.experimental.pallas import tpu_sc as plsc

L = 16
NC = 2
NS = 16
NW = NC * NS

P = 262144
DIM = 64
ORDER = 4
NSEG = 1024
ROW = ORDER * DIM

PW = P // NW
C = 128
NCHUNK = PW // C


def _sc_body(table_hbm, xq_hbm, out_hbm, xq_v, idx_v, t_v, rows_v, out_v, sem):
    wid = lax.axis_index("s") * NC + lax.axis_index("c")
    base = wid * PW

    def chunk_body(k, _):
        off = base + k * C
        pltpu.sync_copy(xq_hbm.at[pl.ds(off, C)], xq_v)
        for v in range(C // L):
            x = xq_v[pl.ds(v * L, L)]
            ix = jnp.clip(x.astype(jnp.int32), 0, NSEG - 1)
            idx_v[pl.ds(v * L, L)] = ix
            t_v[pl.ds(v * L, L)] = x - ix.astype(jnp.float32)
        pltpu.async_copy(table_hbm.at[idx_v], rows_v, sem).wait()

        def grp_body(g, _):
            tvec = t_v[pl.ds(g * L, L)]
            for lane in range(L):
                t = tvec[lane]
                p = g * L + lane
                for q in range(DIM // L):
                    c0 = rows_v[p, pl.ds(0 * DIM + q * L, L)]
                    c1 = rows_v[p, pl.ds(1 * DIM + q * L, L)]
                    c2 = rows_v[p, pl.ds(2 * DIM + q * L, L)]
                    c3 = rows_v[p, pl.ds(3 * DIM + q * L, L)]
                    out_v[p, pl.ds(q * L, L)] = ((c0 * t + c1) * t + c2) * t + c3
            return 0

        lax.fori_loop(0, C // L, grp_body, 0)
        pltpu.sync_copy(out_v, out_hbm.at[pl.ds(off, C)])
        return 0

    lax.fori_loop(0, NCHUNK, chunk_body, 0)


@functools.partial(
    pl.kernel,
    mesh=plsc.VectorSubcoreMesh(core_axis_name="c", subcore_axis_name="s"),
    out_type=jax.ShapeDtypeStruct((P, DIM), jnp.float32),
    scratch_types=[
        pltpu.VMEM((C,), jnp.float32),
        pltpu.VMEM((C,), jnp.int32),
        pltpu.VMEM((C,), jnp.float32),
        pltpu.VMEM((C, ROW), jnp.float32),
        pltpu.VMEM((C, DIM), jnp.float32),
        pltpu.SemaphoreType.DMA,
    ],
)
def _sc_ppoly(table_hbm, xq_hbm, out_hbm, xq_v, idx_v, t_v, rows_v, out_v, sem):
    _sc_body(table_hbm, xq_hbm, out_hbm, xq_v, idx_v, t_v, rows_v, out_v, sem)


def kernel(c, x_breaks, xq, i, j):
    del x_breaks
    table = jnp.transpose(c[:, :, i, j, :], (1, 0, 2)).reshape(NSEG, ROW)
    return _sc_ppoly(table, xq)


# speedup vs baseline: 54.5439x; 1.4124x over previous
import functools

import jax
import jax.numpy as jnp
from jax import lax
from jax.experimental import pallas as pl
from jax.experimental.pallas import tpu as pltpu
from jax.experimental.pallas import tpu_sc as plsc

L = 16
NC = 2
NS = 16
NW = NC * NS

P = 262144
DIM = 64
ORDER = 4
NSEG = 1024
ROW = ORDER * DIM

PW = P // NW
C = 128
NCHUNK = PW // C


def _sc_body(table_hbm, xq_hbm, out_hbm,
             xq_all, idx_all, t_all, rows2, out2, gsem, ssem):
    wid = lax.axis_index("s") * NC + lax.axis_index("c")
    base = wid * PW

    pltpu.sync_copy(xq_hbm.at[pl.ds(base, PW)], xq_all)

    def vt_body(v, _):
        x = xq_all[pl.ds(v * L, L)]
        ix = jnp.clip(x.astype(jnp.int32), 0, NSEG - 1)
        idx_all[pl.ds(v * L, L)] = ix
        t_all[pl.ds(v * L, L)] = x - ix.astype(jnp.float32)
        return 0

    lax.fori_loop(0, PW // L, vt_body, 0)

    def gather(k, buf):
        return pltpu.async_copy(
            table_hbm.at[idx_all.at[pl.ds(k * C, C)]], rows2.at[buf], gsem)

    gather(0, 0)

    def pair_body(s, _):
        for b in range(2):
            k = 2 * s + b
            pltpu.make_async_copy(
                table_hbm.at[idx_all.at[pl.ds(k * C, C)]],
                rows2.at[b], gsem).wait()

            @pl.when(k + 1 < NCHUNK)
            def _():
                gather(k + 1, 1 - b)

            @pl.when(k >= 2)
            def _():
                pltpu.make_async_copy(
                    out2.at[b], out_hbm.at[pl.ds(base, C)], ssem).wait()

            def grp_body(g, _):
                tvec = t_all[pl.ds(k * C + g * L, L)]
                for lane in range(L):
                    t = tvec[lane]
                    p = g * L + lane
                    for q in range(DIM // L):
                        c0 = rows2[b, p, pl.ds(0 * DIM + q * L, L)]
                        c1 = rows2[b, p, pl.ds(1 * DIM + q * L, L)]
                        c2 = rows2[b, p, pl.ds(2 * DIM + q * L, L)]
                        c3 = rows2[b, p, pl.ds(3 * DIM + q * L, L)]
                        out2[b, p, pl.ds(q * L, L)] = (
                            (c0 * t + c1) * t + c2) * t + c3
                return 0

            lax.fori_loop(0, C // L, grp_body, 0)
            pltpu.async_copy(out2.at[b], out_hbm.at[pl.ds(base + k * C, C)],
                             ssem)
        return 0

    lax.fori_loop(0, NCHUNK // 2, pair_body, 0)

    for b in range(2):
        pltpu.make_async_copy(out_hbm.at[pl.ds(base, C)], out2.at[b],
                              ssem).wait()


@functools.partial(
    pl.kernel,
    mesh=plsc.VectorSubcoreMesh(core_axis_name="c", subcore_axis_name="s"),
    out_type=jax.ShapeDtypeStruct((P, DIM), jnp.float32),
    scratch_types=[
        pltpu.VMEM((PW,), jnp.float32),
        pltpu.VMEM((PW,), jnp.int32),
        pltpu.VMEM((PW,), jnp.float32),
        pltpu.VMEM((2, C, ROW), jnp.float32),
        pltpu.VMEM((2, C, DIM), jnp.float32),
        pltpu.SemaphoreType.DMA,
        pltpu.SemaphoreType.DMA,
    ],
)
def _sc_ppoly(table_hbm, xq_hbm, out_hbm,
              xq_all, idx_all, t_all, rows2, out2, gsem, ssem):
    _sc_body(table_hbm, xq_hbm, out_hbm,
             xq_all, idx_all, t_all, rows2, out2, gsem, ssem)


def kernel(c, x_breaks, xq, i, j):
    del x_breaks
    table = jnp.transpose(c[:, :, i, j, :], (1, 0, 2)).reshape(NSEG, ROW)
    return _sc_ppoly(table, xq)


# speedup vs baseline: 82.3717x; 1.5102x over previous
import functools

import jax
import jax.numpy as jnp
from jax import lax
from jax.experimental import pallas as pl
from jax.experimental.pallas import tpu as pltpu
from jax.experimental.pallas import tpu_sc as plsc

L = 16
NC = 2
NS = 16
NW = NC * NS

P = 262144
DIM = 64
ORDER = 4
NSEG = 1024
ROW = ORDER * DIM

PW = P // NW
C = 128
NCHUNK = PW // C


def _sc_body(table_hbm, xq_hbm, out_hbm,
             xq_all, idx_all, t_all, rows2, out2, gsem, ssem):
    wid = lax.axis_index("s") * NC + lax.axis_index("c")
    base = wid * PW

    pltpu.sync_copy(xq_hbm.at[pl.ds(base, PW)], xq_all)

    def vt_body(v, _):
        x = xq_all[pl.ds(v * L, L)]
        ix = jnp.clip(x.astype(jnp.int32), 0, NSEG - 1)
        idx_all[pl.ds(v * L, L)] = ix
        t_all[pl.ds(v * L, L)] = x - ix.astype(jnp.float32)
        return 0

    lax.fori_loop(0, PW // L, vt_body, 0)

    def gather(k, buf):
        return pltpu.async_copy(
            table_hbm.at[idx_all.at[pl.ds(k * C, C)]], rows2.at[buf], gsem)

    gather(0, 0)

    def pair_body(s, _):
        for b in range(2):
            k = 2 * s + b
            pltpu.make_async_copy(
                table_hbm.at[idx_all.at[pl.ds(k * C, C)]],
                rows2.at[b], gsem).wait()

            @pl.when(k + 1 < NCHUNK)
            def _():
                gather(k + 1, 1 - b)

            @pl.when(k >= 2)
            def _():
                pltpu.make_async_copy(
                    out2.at[b], out_hbm.at[pl.ds(base, C)], ssem).wait()

            def grp_body(g, _):
                tvec = t_all[pl.ds(k * C + g * L, L)]
                for lane in range(L):
                    t = tvec[lane]
                    p = g * L + lane
                    nq = DIM // L
                    cs = [[rows2[b, p, pl.ds(m * DIM + q * L, L)]
                           for q in range(nq)] for m in range(ORDER)]
                    t2 = t * t
                    hi = [cs[0][q] * t + cs[1][q] for q in range(nq)]
                    lo = [cs[2][q] * t + cs[3][q] for q in range(nq)]
                    for q in range(nq):
                        out2[b, p, pl.ds(q * L, L)] = hi[q] * t2 + lo[q]
                return 0

            lax.fori_loop(0, C // L, grp_body, 0)
            pltpu.async_copy(out2.at[b], out_hbm.at[pl.ds(base + k * C, C)],
                             ssem)
        return 0

    lax.fori_loop(0, NCHUNK // 2, pair_body, 0)

    for b in range(2):
        pltpu.make_async_copy(out_hbm.at[pl.ds(base, C)], out2.at[b],
                              ssem).wait()


@functools.partial(
    pl.kernel,
    mesh=plsc.VectorSubcoreMesh(core_axis_name="c", subcore_axis_name="s"),
    out_type=jax.ShapeDtypeStruct((P, DIM), jnp.float32),
    scratch_types=[
        pltpu.VMEM((PW,), jnp.float32),
        pltpu.VMEM((PW,), jnp.int32),
        pltpu.VMEM((PW,), jnp.float32),
        pltpu.VMEM((2, C, ROW), jnp.float32),
        pltpu.VMEM((2, C, DIM), jnp.float32),
        pltpu.SemaphoreType.DMA,
        pltpu.SemaphoreType.DMA,
    ],
)
def _sc_ppoly(table_hbm, xq_hbm, out_hbm,
              xq_all, idx_all, t_all, rows2, out2, gsem, ssem):
    _sc_body(table_hbm, xq_hbm, out_hbm,
             xq_all, idx_all, t_all, rows2, out2, gsem, ssem)


def kernel(c, x_breaks, xq, i, j):
    del x_breaks
    table = jnp.transpose(c[:, :, i, j, :], (1, 0, 2)).reshape(NSEG, ROW)
    return _sc_ppoly(table, xq)
